# baseline (speedup 1.0000x reference)
import functools

import jax
import jax.numpy as jnp
from jax import lax
from jax.experimental import pallas as pl
from jax.experimental.pallas import tpu as pltpu
from jax.experimental.pallas import tpu_sc as plsc

V, D, H, O = 1_000_000, 32, 128, 64
B, L = 4096, 200
CHUNK = 16
LP = 208
NCHUNKS = LP // CHUNK
NW = 32
BPW = B // NW


def _pool_body(ia_hbm, ib_hbm, len_hbm, table_hbm, out_hbm,
               idxa_v, idxb_v, len_v, out_v, rows0, rows1, sem0, sem1):
    wid = lax.axis_index("s") * 2 + lax.axis_index("c")
    base = wid * BPW
    pltpu.sync_copy(ia_hbm.at[pl.ds(base, BPW)], idxa_v)
    pltpu.sync_copy(ib_hbm.at[pl.ds(base, BPW)], idxb_v)
    pltpu.sync_copy(len_hbm.at[pl.ds(base, BPW)], len_v)

    HALF = 128 // CHUNK

    def chunk_idx(b, c):
        return lax.cond(
            c < HALF,
            lambda: idxa_v[b, pl.ds(c * CHUNK, CHUNK)],
            lambda: idxb_v[b, pl.ds((c - HALF) * CHUNK, CHUNK)],
        )

    def fire(b, ln, rows, sem):
        nc = lax.div(ln + (CHUNK - 1), CHUNK)

        def body(c, carry):
            pltpu.async_copy(table_hbm.at[chunk_idx(b, c) >> 2], rows.at[c], sem)
            return carry

        lax.fori_loop(0, nc, body, 0)

    def drain_acc_store(b, ln, rows, sem):
        nfull = lax.div(ln, CHUNK)
        rem = ln - nfull * CHUNK
        nc = nfull + jnp.where(rem > 0, 1, 0)
        zeroi = jnp.zeros((16,), jnp.int32)

        def dbody(c, carry):
            pltpu.make_async_copy(table_hbm.at[zeroi], rows.at[0], sem).wait()
            return carry

        lax.fori_loop(0, nc, dbody, 0)

        zero = jnp.zeros((16,), jnp.float32)

        def abody(c, carry):
            a0, a1 = carry
            qv = (chunk_idx(b, c) & 3) * 32
            for t in range(CHUNK):
                q = qv[t]
                a0 = a0 + rows[c, t, pl.ds(q, 16)]
                a1 = a1 + rows[c, t, pl.ds(q + 16, 16)]
            return a0, a1

        a0, a1 = lax.fori_loop(0, nfull, abody, (zero, zero))

        qv = (chunk_idx(b, nfull) & 3) * 32
        for t in range(CHUNK):
            keep = t < rem
            q = qv[t]
            r0 = jnp.where(keep, rows[nfull, t, pl.ds(q, 16)], zero)
            r1 = jnp.where(keep, rows[nfull, t, pl.ds(q + 16, 16)], zero)
            a0 = a0 + r0
            a1 = a1 + r1

        lnv = jnp.full((16,), jnp.maximum(ln.astype(jnp.float32), 1.0))
        out_v[b, 0:16] = a0 / lnv
        out_v[b, 16:32] = a1 / lnv

    G = 16
    NG = BPW // G
    bufs = ((rows0, sem0), (rows1, sem1))

    lvec0 = len_v[pl.ds(0, G)]
    fire(0, lvec0[0], rows0, sem0)

    def outer(g, carry):
        g16 = g * G
        lvec = len_v[pl.ds(g16, G)]
        lvec_next = len_v[pl.ds(jnp.minimum(g16 + G, BPW - G), G)]
        for j in range(G):
            b = g16 + j
            rows_c, sem_c = bufs[j % 2]
            rows_n, sem_n = bufs[(j + 1) % 2]
            if j < G - 1:
                fire(b + 1, lvec[j + 1], rows_n, sem_n)
            else:
                @pl.when(g < NG - 1)
                def _():
                    fire(b + 1, lvec_next[0], rows_n, sem_n)
            drain_acc_store(b, lvec[j], rows_c, sem_c)
        return carry

    lax.fori_loop(0, NG, outer, 0)
    pltpu.sync_copy(out_v, out_hbm.at[pl.ds(base, BPW)])


_pool = functools.partial(
    pl.kernel,
    out_type=jax.ShapeDtypeStruct((B, D), jnp.float32),
    mesh=plsc.VectorSubcoreMesh(
        core_axis_name="c", subcore_axis_name="s", num_cores=2, num_subcores=16
    ),
    scratch_types=[
        pltpu.VMEM((BPW, 128), jnp.int32),
        pltpu.VMEM((BPW, 128), jnp.int32),
        pltpu.VMEM((BPW,), jnp.int32),
        pltpu.VMEM((BPW, D), jnp.float32),
        pltpu.VMEM((NCHUNKS, CHUNK, 128), jnp.float32),
        pltpu.VMEM((NCHUNKS, CHUNK, 128), jnp.float32),
        pltpu.SemaphoreType.DMA,
        pltpu.SemaphoreType.DMA,
    ],
    compiler_params=pltpu.CompilerParams(
        use_tc_tiling_on_sc=True, needs_layout_passes=False
    ),
)(_pool_body)


def _prep_body(x_ref, a_ref, b_ref):
    xt = jnp.transpose(x_ref[...])
    a_ref[...] = xt[:, 0:128]
    b_ref[...] = xt[:, 128:256]


def _prep(xp):
    CB = 512
    return pl.pallas_call(
        _prep_body,
        grid=(B // CB,),
        in_specs=[pl.BlockSpec((256, CB), lambda i: (0, i))],
        out_specs=[
            pl.BlockSpec((CB, 128), lambda i: (i, 0)),
            pl.BlockSpec((CB, 128), lambda i: (i, 0)),
        ],
        out_shape=[
            jax.ShapeDtypeStruct((B, 128), jnp.int32),
            jax.ShapeDtypeStruct((B, 128), jnp.int32),
        ],
    )(xp)


def _ffnn_body(vec_ref, w1_ref, b1_ref, w2_ref, b2_ref, out_ref):
    x = vec_ref[...]
    h = jnp.maximum(
        jnp.dot(x, w1_ref[...], preferred_element_type=jnp.float32) + b1_ref[...],
        0.0,
    )
    lg = jnp.dot(h, w2_ref[...], preferred_element_type=jnp.float32) + b2_ref[...]
    m = jnp.max(lg, axis=1, keepdims=True)
    ex = jnp.exp(lg - m)
    out_ref[...] = lg - m - jnp.log(jnp.sum(ex, axis=1, keepdims=True))


def _ffnn(vec, W1, b1, W2, b2):
    RB = 512
    return pl.pallas_call(
        _ffnn_body,
        grid=(B // RB,),
        in_specs=[
            pl.BlockSpec((RB, D), lambda i: (i, 0)),
            pl.BlockSpec((D, H), lambda i: (0, 0)),
            pl.BlockSpec((1, H), lambda i: (0, 0)),
            pl.BlockSpec((H, O), lambda i: (0, 0)),
            pl.BlockSpec((1, O), lambda i: (0, 0)),
        ],
        out_specs=pl.BlockSpec((RB, O), lambda i: (i, 0)),
        out_shape=jax.ShapeDtypeStruct((B, O), jnp.float32),
    )(vec, W1, b1.reshape(1, H), W2, b2.reshape(1, O))


def kernel(input, lengths, table, W1, b1, W2, b2):
    xp = jnp.pad(input, ((0, 256 - L), (0, 0)))
    ia, ib = _prep(xp)
    vec = _pool(ia, ib, lengths, table.reshape(V // 4, 128))
    return _ffnn(vec, W1, b1, W2, b2)


# speedup vs baseline: 1.1107x; 1.1107x over previous
import functools

import jax
import jax.numpy as jnp
from jax import lax
from jax.experimental import pallas as pl
from jax.experimental.pallas import tpu as pltpu
from jax.experimental.pallas import tpu_sc as plsc

V, D, H, O = 1_000_000, 32, 128, 64
B, L = 4096, 200
CHUNK = 16
LP = 208
NCHUNKS = LP // CHUNK
NW = 32
BPW = B // NW


def _pool_body(ia_hbm, ib_hbm, len_hbm, table_hbm, out_hbm,
               idxa_v, idxb_v, len_v, out_v, rows0, rows1, rows2, rows3,
               sem0, sem1, sem2, sem3):
    wid = lax.axis_index("s") * 2 + lax.axis_index("c")
    base = wid * BPW
    pltpu.sync_copy(ia_hbm.at[pl.ds(base, BPW)], idxa_v)
    pltpu.sync_copy(ib_hbm.at[pl.ds(base, BPW)], idxb_v)
    pltpu.sync_copy(len_hbm.at[pl.ds(base, BPW)], len_v)

    HALF = 128 // CHUNK

    def fire(b, ln, rows, sem):
        nc = lax.div(ln + (CHUNK - 1), CHUNK)
        nca = jnp.minimum(nc, HALF)

        def body_a(c, carry):
            idx = idxa_v.at[b, pl.ds(c * CHUNK, CHUNK)]
            pltpu.async_copy(table_hbm.at[idx], rows.at[c], sem)
            return carry

        def body_b(c, carry):
            idx = idxb_v.at[b, pl.ds((c - HALF) * CHUNK, CHUNK)]
            pltpu.async_copy(table_hbm.at[idx], rows.at[c], sem)
            return carry

        lax.fori_loop(0, nca, body_a, 0)
        lax.fori_loop(HALF, nc, body_b, 0)

    def drain_acc_store(b, ln, rows, sem):
        nfull = lax.div(ln, CHUNK)
        rem = ln - nfull * CHUNK
        nc = nfull + jnp.where(rem > 0, 1, 0)

        def dbody(c, carry):
            pltpu.make_async_copy(
                table_hbm.at[idxa_v.at[0, pl.ds(0, CHUNK)]], rows.at[0], sem
            ).wait()
            return carry

        lax.fori_loop(0, nc, dbody, 0)

        zero = jnp.zeros((16,), jnp.float32)

        def abody(c, carry):
            a0, a1 = carry
            for t in range(CHUNK):
                a0 = a0 + rows[c, t, 0:16]
                a1 = a1 + rows[c, t, 16:32]
            return a0, a1

        a0, a1 = lax.fori_loop(0, nfull, abody, (zero, zero))

        for t in range(CHUNK):
            keep = t < rem
            r0 = jnp.where(keep, rows[nfull, t, 0:16], zero)
            r1 = jnp.where(keep, rows[nfull, t, 16:32], zero)
            a0 = a0 + r0
            a1 = a1 + r1

        lnv = jnp.full((16,), jnp.maximum(ln.astype(jnp.float32), 1.0))
        out_v[b, 0:16] = a0 / lnv
        out_v[b, 16:32] = a1 / lnv

    G = 16
    NG = BPW // G
    AHEAD = 3
    bufs = ((rows0, sem0), (rows1, sem1), (rows2, sem2), (rows3, sem3))

    lvec0 = len_v[pl.ds(0, G)]
    for k in range(AHEAD):
        fire(k, lvec0[k], *bufs[k])

    def outer(g, carry):
        g16 = g * G
        lvec = len_v[pl.ds(g16, G)]
        lvec_next = len_v[pl.ds(jnp.minimum(g16 + G, BPW - G), G)]
        for j in range(G):
            b = g16 + j
            ln_ahead = lvec[j + AHEAD] if j < G - AHEAD else lvec_next[j - (G - AHEAD)]
            rows_n, sem_n = bufs[(j + AHEAD) % 4]

            @pl.when(b + AHEAD < BPW)
            def _():
                fire(b + AHEAD, ln_ahead, rows_n, sem_n)

            drain_acc_store(b, lvec[j], *bufs[j % 4])
        return carry

    lax.fori_loop(0, NG, outer, 0)
    pltpu.sync_copy(out_v, out_hbm.at[pl.ds(base, BPW)])


_pool = functools.partial(
    pl.kernel,
    out_type=jax.ShapeDtypeStruct((B, D), jnp.float32),
    mesh=plsc.VectorSubcoreMesh(
        core_axis_name="c", subcore_axis_name="s", num_cores=2, num_subcores=16
    ),
    scratch_types=[
        pltpu.VMEM((BPW, 128), jnp.int32),
        pltpu.VMEM((BPW, 128), jnp.int32),
        pltpu.VMEM((BPW,), jnp.int32),
        pltpu.VMEM((BPW, D), jnp.float32),
        pltpu.VMEM((NCHUNKS, CHUNK, D), jnp.float32),
        pltpu.VMEM((NCHUNKS, CHUNK, D), jnp.float32),
        pltpu.VMEM((NCHUNKS, CHUNK, D), jnp.float32),
        pltpu.VMEM((NCHUNKS, CHUNK, D), jnp.float32),
        pltpu.SemaphoreType.DMA,
        pltpu.SemaphoreType.DMA,
        pltpu.SemaphoreType.DMA,
        pltpu.SemaphoreType.DMA,
    ],
    compiler_params=pltpu.CompilerParams(
        use_tc_tiling_on_sc=False, needs_layout_passes=False
    ),
)(_pool_body)


def _prep_body(x_ref, a_ref, b_ref):
    xt = jnp.transpose(x_ref[...])
    a_ref[...] = xt[:, 0:128]
    b_ref[...] = xt[:, 128:256]


def _prep(xp):
    CB = 512
    return pl.pallas_call(
        _prep_body,
        grid=(B // CB,),
        in_specs=[pl.BlockSpec((256, CB), lambda i: (0, i))],
        out_specs=[
            pl.BlockSpec((CB, 128), lambda i: (i, 0)),
            pl.BlockSpec((CB, 128), lambda i: (i, 0)),
        ],
        out_shape=[
            jax.ShapeDtypeStruct((B, 128), jnp.int32),
            jax.ShapeDtypeStruct((B, 128), jnp.int32),
        ],
    )(xp)


def _ffnn_body(vec_ref, w1_ref, b1_ref, w2_ref, b2_ref, out_ref):
    x = vec_ref[...]
    h = jnp.maximum(
        jnp.dot(x, w1_ref[...], preferred_element_type=jnp.float32) + b1_ref[...],
        0.0,
    )
    lg = jnp.dot(h, w2_ref[...], preferred_element_type=jnp.float32) + b2_ref[...]
    m = jnp.max(lg, axis=1, keepdims=True)
    ex = jnp.exp(lg - m)
    out_ref[...] = lg - m - jnp.log(jnp.sum(ex, axis=1, keepdims=True))


def _ffnn(vec, W1, b1, W2, b2):
    RB = 512
    return pl.pallas_call(
        _ffnn_body,
        grid=(B // RB,),
        in_specs=[
            pl.BlockSpec((RB, D), lambda i: (i, 0)),
            pl.BlockSpec((D, H), lambda i: (0, 0)),
            pl.BlockSpec((1, H), lambda i: (0, 0)),
            pl.BlockSpec((H, O), lambda i: (0, 0)),
            pl.BlockSpec((1, O), lambda i: (0, 0)),
        ],
        out_specs=pl.BlockSpec((RB, O), lambda i: (i, 0)),
        out_shape=jax.ShapeDtypeStruct((B, O), jnp.float32),
    )(vec, W1, b1.reshape(1, H), W2, b2.reshape(1, O))


def kernel(input, lengths, table, W1, b1, W2, b2):
    xp = jnp.pad(input, ((0, 256 - L), (0, 0)))
    ia, ib = _prep(xp)
    vec = _pool(ia, ib, lengths, table)
    return _ffnn(vec, W1, b1, W2, b2)
